# baseline (speedup 1.0000x reference)
import functools

import jax
import jax.numpy as jnp
from jax import lax
from jax.experimental import pallas as pl
from jax.experimental.pallas import tpu as pltpu
from jax.experimental.pallas import tpu_sc as plsc

H = 128
N_MESH = 10000
N_GRID = 10000
N_HALO = 2048
E = 320000

NC = 2
NS = 16
NW = NC * NS
GBLK = 80
NCHUNK = 5
CHUNK = E // NCHUNK
EPWC = CHUNK // NW
NBLKC = EPWC // GBLK

NBUF = 5
SNBUF = 3


@functools.cache
def _sc_mesh():
    return plsc.VectorSubcoreMesh(core_axis_name="c", subcore_axis_name="s",
                                  num_cores=NC, num_subcores=NS)


def _ab_body(mesh_ref, grid_ref, w1a_ref, w1b_ref, a_ref, b_ref):
    a_ref[...] = jnp.dot(mesh_ref[...], w1a_ref[...],
                         preferred_element_type=jnp.float32)
    b_ref[...] = jnp.dot(grid_ref[...], w1b_ref[...],
                         preferred_element_type=jnp.float32)


def _compute_ab(mesh, grid, w1a, w1b):
    blk = 2000
    nblk = N_MESH // blk
    return pl.pallas_call(
        _ab_body,
        grid=(nblk,),
        in_specs=[
            pl.BlockSpec((blk, H), lambda i: (i, 0)),
            pl.BlockSpec((blk, H), lambda i: (i, 0)),
            pl.BlockSpec((H, H), lambda i: (0, 0)),
            pl.BlockSpec((H, H), lambda i: (0, 0)),
        ],
        out_specs=[
            pl.BlockSpec((blk, H), lambda i: (i, 0)),
            pl.BlockSpec((blk, H), lambda i: (i, 0)),
        ],
        out_shape=[
            jax.ShapeDtypeStruct((N_MESH, H), jnp.float32),
            jax.ShapeDtypeStruct((N_GRID, H), jnp.float32),
        ],
    )(mesh, grid, w1a, w1b)


def _make_gather_body():
  def _gather_body(a_hbm, b_hbm, halo_hbm, src_hbm, dst_hbm, out_hbm,
                     halo_v, src_v, dst_v, rows_v, sem_i, sem_a, sem_b, sem_w):
      wid = lax.axis_index("s") * NC + lax.axis_index("c")
      pltpu.sync_copy(halo_hbm, halo_v)
      in0 = wid * EPWC
      base0 = wid * EPWC

      def issue_idx(i, b):
          base = in0 + i * GBLK
          pltpu.async_copy(src_hbm.at[pl.ds(base, GBLK)], src_v.at[b],
                           sem_i.at[b])
          pltpu.async_copy(dst_hbm.at[pl.ds(base, GBLK)], dst_v.at[b],
                           sem_i.at[b])

      def wait_idx(b):
          pltpu.make_async_copy(src_hbm.at[pl.ds(0, GBLK)], src_v.at[b],
                                sem_i.at[b]).wait()
          pltpu.make_async_copy(dst_hbm.at[pl.ds(0, GBLK)], dst_v.at[b],
                                sem_i.at[b]).wait()

      def remap(b):
          def step(j, c):
              s = src_v[b, pl.ds(j * 16, 16)]
              m = s >= N_MESH
              hidx = jnp.where(m, s - N_MESH, 0)
              hv = plsc.load_gather(halo_v, [hidx])
              src_v[b, pl.ds(j * 16, 16)] = jnp.where(m, hv, s)
              return c

          lax.fori_loop(0, GBLK // 16, step, 0, unroll=True)

      def wait_rows(b, sem):
          pltpu.make_async_copy(a_hbm.at[src_v.at[b]], rows_v.at[b],
                                sem.at[b]).wait()

      issue_idx(0, 0)
      issue_idx(1, 1)

      def step(i, carry):
          b = lax.rem(i, NBUF)
          wait_idx(b)
          remap(b)

          @pl.when(i >= NBUF)
          def _():
              pltpu.make_async_copy(rows_v.at[b],
                                    out_hbm.at[pl.ds(base0, GBLK)],
                                    sem_w.at[b]).wait()

          pltpu.async_copy(a_hbm.at[src_v.at[b]], rows_v.at[b], sem_a.at[b])

          @pl.when(i >= 1)
          def _():
              b1 = lax.rem(i - 1, NBUF)
              wait_rows(b1, sem_a)
              pltpu.async_copy(b_hbm.at[dst_v.at[b1]], rows_v.at[b1],
                               sem_b.at[b1], add=True)

          @pl.when(i >= 2)
          def _():
              b2 = lax.rem(i - 2, NBUF)
              wait_rows(b2, sem_b)
              base2 = base0 + (i - 2) * GBLK
              pltpu.async_copy(rows_v.at[b2], out_hbm.at[pl.ds(base2, GBLK)],
                               sem_w.at[b2])

          @pl.when(i + 2 < NBLKC)
          def _():
              issue_idx(i + 2, lax.rem(i + 2, NBUF))
          return carry

      lax.fori_loop(0, NBLKC, step, 0)

      bL = (NBLKC - 1) % NBUF
      b2 = (NBLKC - 2) % NBUF
      wait_rows(bL, sem_a)
      pltpu.async_copy(b_hbm.at[dst_v.at[bL]], rows_v.at[bL], sem_b.at[bL],
                       add=True)
      wait_rows(b2, sem_b)
      pltpu.async_copy(rows_v.at[b2],
                       out_hbm.at[pl.ds(base0 + (NBLKC - 2) * GBLK, GBLK)],
                       sem_w.at[b2])
      wait_rows(bL, sem_b)
      pltpu.async_copy(rows_v.at[bL],
                       out_hbm.at[pl.ds(base0 + (NBLKC - 1) * GBLK, GBLK)],
                       sem_w.at[bL])
      for b in range(NBUF):
          pltpu.make_async_copy(rows_v.at[b], out_hbm.at[pl.ds(base0, GBLK)],
                                sem_w.at[b]).wait()


  return _gather_body


@functools.cache
def _gather_call():
    return pl.kernel(
        _make_gather_body(),
        out_type=jax.ShapeDtypeStruct((CHUNK, H), jnp.float32),
        mesh=_sc_mesh(),
        compiler_params=pltpu.CompilerParams(needs_layout_passes=False),
        scratch_types=[
            pltpu.VMEM((N_HALO,), jnp.int32),
            pltpu.VMEM((NBUF, GBLK), jnp.int32),
            pltpu.VMEM((NBUF, GBLK), jnp.int32),
            pltpu.VMEM((NBUF, GBLK, H), jnp.float32),
            pltpu.SemaphoreType.DMA((NBUF,)),
            pltpu.SemaphoreType.DMA((NBUF,)),
            pltpu.SemaphoreType.DMA((NBUF,)),
            pltpu.SemaphoreType.DMA((NBUF,)),
        ],
    )


def _edge_body(x_ref, g_ref, w1c_ref, w2_ref, b1_ref, b2_ref, g_ln_ref,
               b_ln_ref, o_ref):
    x = x_ref[...]
    h = jnp.dot(x, w1c_ref[...], preferred_element_type=jnp.float32)
    h = h + g_ref[...] + b1_ref[...]
    h = h * jax.nn.sigmoid(h)
    h = jnp.dot(h, w2_ref[...], preferred_element_type=jnp.float32)
    h = h + b2_ref[...]
    m = jnp.mean(h, axis=-1, keepdims=True)
    d = h - m
    v = jnp.mean(d * d, axis=-1, keepdims=True)
    ln = d * lax.rsqrt(v + 1e-5) * g_ln_ref[...] + b_ln_ref[...]
    o_ref[...] = x + ln


_EBLK = 3200
_EBPC = CHUNK // _EBLK


def _edge_mlp(c, edge, g1, w1c, w2, b1, b2, g_ln, b_ln):
    cb0 = c * _EBPC
    wspec = pl.BlockSpec((H, H), lambda i: (0, 0))
    vspec = pl.BlockSpec((1, H), lambda i: (0, 0))
    return pl.pallas_call(
        _edge_body,
        grid=(_EBPC,),
        in_specs=[
            pl.BlockSpec((_EBLK, H), lambda i: (cb0 + i, 0)),
            pl.BlockSpec((_EBLK, H), lambda i: (i, 0)),
            wspec, wspec, vspec, vspec, vspec, vspec,
        ],
        out_specs=pl.BlockSpec((_EBLK, H), lambda i: (i, 0)),
        out_shape=jax.ShapeDtypeStruct((CHUNK, H), jnp.float32),
    )(edge, g1, w1c, w2, b1, b2, g_ln, b_ln)


def _scatter_body(ef0, ef1, ef2, ef3, ef4, dst_hbm, out_hbm,
                  idx_v, rows_v, zero_v, acc_sh, sem_l, sem_s):
    cid = lax.axis_index("c")
    sid = lax.axis_index("s")
    wid = sid * NC + cid

    def zrow(r, c):
        def zcol(q, c2):
            zero_v[r, pl.ds(q * 16, 16)] = jnp.zeros((16,), jnp.float32)
            return c2
        return lax.fori_loop(0, H // 16, zcol, c, unroll=True)

    lax.fori_loop(0, 16, zrow, 0)

    nb16 = N_GRID // 16

    def zput(t, c):
        bid = t * NS + sid

        @pl.when(bid < nb16)
        def _():
            pltpu.async_copy(zero_v, acc_sh.at[pl.ds(bid * 16, 16)],
                             sem_l.at[0])
        return c

    def zdrain(t, c):
        bid = t * NS + sid

        @pl.when(bid < nb16)
        def _():
            pltpu.make_async_copy(zero_v, acc_sh.at[pl.ds(0, 16)],
                                  sem_l.at[0]).wait()
        return c

    lax.fori_loop(0, (nb16 + NS - 1) // NS, zput, 0)
    lax.fori_loop(0, (nb16 + NS - 1) // NS, zdrain, 0)
    plsc.subcore_barrier()

    for c, ef_hbm in enumerate((ef0, ef1, ef2, ef3, ef4)):
        in0 = c * CHUNK + wid * EPWC
        ef0w = wid * EPWC

        def issue_load(i, b, ef_hbm=ef_hbm, in0=in0, ef0w=ef0w):
            pltpu.async_copy(dst_hbm.at[pl.ds(in0 + i * GBLK, GBLK)],
                             idx_v.at[b], sem_l.at[b])
            pltpu.async_copy(ef_hbm.at[pl.ds(ef0w + i * GBLK, GBLK)],
                             rows_v.at[b], sem_l.at[b])

        def wait_load(b, ef_hbm=ef_hbm):
            pltpu.make_async_copy(dst_hbm.at[pl.ds(0, GBLK)], idx_v.at[b],
                                  sem_l.at[b]).wait()
            pltpu.make_async_copy(ef_hbm.at[pl.ds(0, GBLK)], rows_v.at[b],
                                  sem_l.at[b]).wait()

        def wait_scat(b):
            pltpu.make_async_copy(rows_v.at[b], acc_sh.at[idx_v.at[b]],
                                  sem_s.at[b]).wait()

        for j in range(SNBUF - 1):
            issue_load(j, j)

        def block(i, carry, issue_load=issue_load, wait_load=wait_load,
                  wait_scat=wait_scat):
            b = lax.rem(i, SNBUF)
            wait_load(b)
            pltpu.async_copy(rows_v.at[b], acc_sh.at[idx_v.at[b]],
                             sem_s.at[b], add=True)

            @pl.when(i + SNBUF - 1 < NBLKC)
            def _():
                bn = lax.rem(i + SNBUF - 1, SNBUF)

                @pl.when(i >= 1)
                def _():
                    wait_scat(bn)
                issue_load(i + SNBUF - 1, bn)
            return carry

        lax.fori_loop(0, NBLKC, block, 0)
        for b in range(SNBUF):
            wait_scat(b)

    plsc.subcore_barrier()

    def put(t, c):
        bid = t * NS + sid

        @pl.when(bid < nb16)
        def _():
            b = lax.rem(t, SNBUF)

            @pl.when(t >= SNBUF)
            def _():
                pltpu.make_async_copy(rows_v.at[b, pl.ds(0, 16)],
                                      out_hbm.at[cid, pl.ds(0, 16)],
                                      sem_s.at[b]).wait()
            pltpu.sync_copy(acc_sh.at[pl.ds(bid * 16, 16)],
                            rows_v.at[b, pl.ds(0, 16)])
            pltpu.async_copy(rows_v.at[b, pl.ds(0, 16)],
                             out_hbm.at[cid, pl.ds(bid * 16, 16)],
                             sem_s.at[b])
        return c

    lax.fori_loop(0, (nb16 + NS - 1) // NS, put, 0)

    ntb_tile = lax.div(nb16 - sid + NS - 1, NS)

    def pdrain(t, c):
        bid = t * NS + sid

        @pl.when(jnp.logical_and(bid < nb16, t + SNBUF >= ntb_tile))
        def _():
            b = lax.rem(t, SNBUF)
            pltpu.make_async_copy(rows_v.at[b, pl.ds(0, 16)],
                                  out_hbm.at[cid, pl.ds(0, 16)],
                                  sem_s.at[b]).wait()
        return c

    lax.fori_loop(0, (nb16 + NS - 1) // NS, pdrain, 0)


@functools.cache
def _scatter_call():
    return pl.kernel(
        _scatter_body,
        out_type=jax.ShapeDtypeStruct((NC, N_GRID, H), jnp.float32),
        mesh=_sc_mesh(),
        compiler_params=pltpu.CompilerParams(needs_layout_passes=False),
        scratch_types=[
            pltpu.VMEM((SNBUF, GBLK), jnp.int32),
            pltpu.VMEM((SNBUF, GBLK, H), jnp.float32),
            pltpu.VMEM((16, H), jnp.float32),
            pltpu.VMEM_SHARED((N_GRID, H), jnp.float32),
            pltpu.SemaphoreType.DMA((SNBUF,)),
            pltpu.SemaphoreType.DMA((SNBUF,)),
        ],
    )


def _node_body(grid_ref, a0_ref, a1_ref, w1a_ref, w1b_ref, w2_ref, b1_ref,
               b2_ref, g_ln_ref, b_ln_ref, mask_ref, o_ref):
    x = grid_ref[...]
    ag = a0_ref[...] + a1_ref[...]
    h = jnp.dot(x, w1a_ref[...], preferred_element_type=jnp.float32)
    h = h + jnp.dot(ag, w1b_ref[...], preferred_element_type=jnp.float32)
    h = h + b1_ref[...]
    h = h * jax.nn.sigmoid(h)
    h = jnp.dot(h, w2_ref[...], preferred_element_type=jnp.float32)
    h = h + b2_ref[...]
    m = jnp.mean(h, axis=-1, keepdims=True)
    d = h - m
    v = jnp.mean(d * d, axis=-1, keepdims=True)
    ln = d * lax.rsqrt(v + 1e-5) * g_ln_ref[...] + b_ln_ref[...]
    o_ref[...] = x + ln * mask_ref[...]


def _node_mlp(grid, a0, a1, w1a, w1b, w2, b1, b2, g_ln, b_ln, mask):
    blk = 2000
    nblk = N_GRID // blk
    wspec = pl.BlockSpec((H, H), lambda i: (0, 0))
    vspec = pl.BlockSpec((1, H), lambda i: (0, 0))
    bspec = pl.BlockSpec((blk, H), lambda i: (i, 0))
    return pl.pallas_call(
        _node_body,
        grid=(nblk,),
        in_specs=[bspec, bspec, bspec, wspec, wspec, wspec, vspec, vspec,
                  vspec, vspec, pl.BlockSpec((blk, 1), lambda i: (i, 0))],
        out_specs=bspec,
        out_shape=jax.ShapeDtypeStruct((N_GRID, H), jnp.float32),
    )(grid, a0, a1, w1a, w1b, w2, b1, b2, g_ln, b_ln, mask)


def kernel(mesh2grid_edge_features, grid_node_features, mesh_node_features,
           halo_idx, dst_indices, src_indices, num_local,
           eW1, eb1, eW2, eb2, eg, ebb,
           nW1, nb1, nW2, nb2, ng, nbb):
    a_tab, b_tab = _compute_ab(mesh_node_features, grid_node_features,
                               eW1[:H], eW1[H:2 * H])

    w1c = eW1[2 * H:]
    eb1r, eb2r = eb1.reshape(1, H), eb2.reshape(1, H)
    egr, ebbr = eg.reshape(1, H), ebb.reshape(1, H)
    e_chunks = []
    for c in range(NCHUNK):
        sl = slice(c * CHUNK, (c + 1) * CHUNK)
        g1 = _gather_call()(a_tab, b_tab, halo_idx, src_indices[sl],
                            dst_indices[sl])
        e_chunks.append(_edge_mlp(c, mesh2grid_edge_features, g1, w1c, eW2,
                                  eb1r, eb2r, egr, ebbr))

    agg = _scatter_call()(*e_chunks, dst_indices)

    mask = (jnp.arange(N_GRID, dtype=jnp.int32)[:, None]
            < num_local).astype(jnp.float32)
    return _node_mlp(grid_node_features, agg[0], agg[1],
                     nW1[:H], nW1[H:], nW2,
                     nb1.reshape(1, H), nb2.reshape(1, H),
                     ng.reshape(1, H), nbb.reshape(1, H), mask)


# speedup vs baseline: 1.0015x; 1.0015x over previous
import functools

import jax
import jax.numpy as jnp
from jax import lax
from jax.experimental import pallas as pl
from jax.experimental.pallas import tpu as pltpu
from jax.experimental.pallas import tpu_sc as plsc

H = 128
N_MESH = 10000
N_GRID = 10000
N_HALO = 2048
E = 320000

NC = 2
NS = 16
NW = NC * NS
GBLK = 80
NCHUNK = 5
CHUNK = E // NCHUNK
EPWC = CHUNK // NW
NBLKC = EPWC // GBLK

NBUF = 5
SNBUF = 3


@functools.cache
def _sc_mesh():
    return plsc.VectorSubcoreMesh(core_axis_name="c", subcore_axis_name="s",
                                  num_cores=NC, num_subcores=NS)


def _ab_body(mesh_ref, grid_ref, w1a_ref, w1b_ref, a_ref, b_ref):
    a_ref[...] = jnp.dot(mesh_ref[...], w1a_ref[...],
                         preferred_element_type=jnp.float32)
    b_ref[...] = jnp.dot(grid_ref[...], w1b_ref[...],
                         preferred_element_type=jnp.float32)


def _compute_ab(mesh, grid, w1a, w1b):
    blk = 2000
    nblk = N_MESH // blk
    return pl.pallas_call(
        _ab_body,
        grid=(nblk,),
        in_specs=[
            pl.BlockSpec((blk, H), lambda i: (i, 0)),
            pl.BlockSpec((blk, H), lambda i: (i, 0)),
            pl.BlockSpec((H, H), lambda i: (0, 0)),
            pl.BlockSpec((H, H), lambda i: (0, 0)),
        ],
        out_specs=[
            pl.BlockSpec((blk, H), lambda i: (i, 0)),
            pl.BlockSpec((blk, H), lambda i: (i, 0)),
        ],
        out_shape=[
            jax.ShapeDtypeStruct((N_MESH, H), jnp.float32),
            jax.ShapeDtypeStruct((N_GRID, H), jnp.float32),
        ],
    )(mesh, grid, w1a, w1b)


def _make_gather_body(coff):
  def _gather_body(a_hbm, b_hbm, halo_hbm, src_hbm, dst_hbm, out_hbm,
                     halo_v, src_v, dst_v, rows_v, sem_i, sem_a, sem_b, sem_w):
      wid = lax.axis_index("s") * NC + lax.axis_index("c")
      pltpu.sync_copy(halo_hbm, halo_v)
      in0 = coff + wid * EPWC
      base0 = wid * EPWC

      def issue_idx(i, b):
          base = in0 + i * GBLK
          pltpu.async_copy(src_hbm.at[pl.ds(base, GBLK)], src_v.at[b],
                           sem_i.at[b])
          pltpu.async_copy(dst_hbm.at[pl.ds(base, GBLK)], dst_v.at[b],
                           sem_i.at[b])

      def wait_idx(b):
          pltpu.make_async_copy(src_hbm.at[pl.ds(0, GBLK)], src_v.at[b],
                                sem_i.at[b]).wait()
          pltpu.make_async_copy(dst_hbm.at[pl.ds(0, GBLK)], dst_v.at[b],
                                sem_i.at[b]).wait()

      def remap(b):
          def step(j, c):
              s = src_v[b, pl.ds(j * 16, 16)]
              m = s >= N_MESH
              hidx = jnp.where(m, s - N_MESH, 0)
              hv = plsc.load_gather(halo_v, [hidx])
              src_v[b, pl.ds(j * 16, 16)] = jnp.where(m, hv, s)
              return c

          lax.fori_loop(0, GBLK // 16, step, 0, unroll=True)

      def wait_rows(b, sem):
          pltpu.make_async_copy(a_hbm.at[src_v.at[b]], rows_v.at[b],
                                sem.at[b]).wait()

      issue_idx(0, 0)
      issue_idx(1, 1)

      def step(i, carry):
          b = lax.rem(i, NBUF)
          wait_idx(b)
          remap(b)

          @pl.when(i >= NBUF)
          def _():
              pltpu.make_async_copy(rows_v.at[b],
                                    out_hbm.at[pl.ds(base0, GBLK)],
                                    sem_w.at[b]).wait()

          pltpu.async_copy(a_hbm.at[src_v.at[b]], rows_v.at[b], sem_a.at[b])

          @pl.when(i >= 1)
          def _():
              b1 = lax.rem(i - 1, NBUF)
              wait_rows(b1, sem_a)
              pltpu.async_copy(b_hbm.at[dst_v.at[b1]], rows_v.at[b1],
                               sem_b.at[b1], add=True)

          @pl.when(i >= 2)
          def _():
              b2 = lax.rem(i - 2, NBUF)
              wait_rows(b2, sem_b)
              base2 = base0 + (i - 2) * GBLK
              pltpu.async_copy(rows_v.at[b2], out_hbm.at[pl.ds(base2, GBLK)],
                               sem_w.at[b2])

          @pl.when(i + 2 < NBLKC)
          def _():
              issue_idx(i + 2, lax.rem(i + 2, NBUF))
          return carry

      lax.fori_loop(0, NBLKC, step, 0)

      bL = (NBLKC - 1) % NBUF
      b2 = (NBLKC - 2) % NBUF
      wait_rows(bL, sem_a)
      pltpu.async_copy(b_hbm.at[dst_v.at[bL]], rows_v.at[bL], sem_b.at[bL],
                       add=True)
      wait_rows(b2, sem_b)
      pltpu.async_copy(rows_v.at[b2],
                       out_hbm.at[pl.ds(base0 + (NBLKC - 2) * GBLK, GBLK)],
                       sem_w.at[b2])
      wait_rows(bL, sem_b)
      pltpu.async_copy(rows_v.at[bL],
                       out_hbm.at[pl.ds(base0 + (NBLKC - 1) * GBLK, GBLK)],
                       sem_w.at[bL])
      for b in range(NBUF):
          pltpu.make_async_copy(rows_v.at[b], out_hbm.at[pl.ds(base0, GBLK)],
                                sem_w.at[b]).wait()


  return _gather_body


@functools.cache
def _gather_call(c):
    return pl.kernel(
        _make_gather_body(c * CHUNK),
        out_type=jax.ShapeDtypeStruct((CHUNK, H), jnp.float32),
        mesh=_sc_mesh(),
        compiler_params=pltpu.CompilerParams(needs_layout_passes=False),
        scratch_types=[
            pltpu.VMEM((N_HALO,), jnp.int32),
            pltpu.VMEM((NBUF, GBLK), jnp.int32),
            pltpu.VMEM((NBUF, GBLK), jnp.int32),
            pltpu.VMEM((NBUF, GBLK, H), jnp.float32),
            pltpu.SemaphoreType.DMA((NBUF,)),
            pltpu.SemaphoreType.DMA((NBUF,)),
            pltpu.SemaphoreType.DMA((NBUF,)),
            pltpu.SemaphoreType.DMA((NBUF,)),
        ],
    )


def _edge_body(x_ref, g_ref, w1c_ref, w2_ref, b1_ref, b2_ref, g_ln_ref,
               b_ln_ref, o_ref):
    x = x_ref[...]
    h = jnp.dot(x, w1c_ref[...], preferred_element_type=jnp.float32)
    h = h + g_ref[...] + b1_ref[...]
    h = h * jax.nn.sigmoid(h)
    h = jnp.dot(h, w2_ref[...], preferred_element_type=jnp.float32)
    h = h + b2_ref[...]
    m = jnp.mean(h, axis=-1, keepdims=True)
    d = h - m
    v = jnp.mean(d * d, axis=-1, keepdims=True)
    ln = d * lax.rsqrt(v + 1e-5) * g_ln_ref[...] + b_ln_ref[...]
    o_ref[...] = x + ln


_EBLK = 3200
_EBPC = CHUNK // _EBLK


def _edge_mlp(c, edge, g1, w1c, w2, b1, b2, g_ln, b_ln):
    cb0 = c * _EBPC
    wspec = pl.BlockSpec((H, H), lambda i: (0, 0))
    vspec = pl.BlockSpec((1, H), lambda i: (0, 0))
    return pl.pallas_call(
        _edge_body,
        grid=(_EBPC,),
        in_specs=[
            pl.BlockSpec((_EBLK, H), lambda i: (cb0 + i, 0)),
            pl.BlockSpec((_EBLK, H), lambda i: (i, 0)),
            wspec, wspec, vspec, vspec, vspec, vspec,
        ],
        out_specs=pl.BlockSpec((_EBLK, H), lambda i: (i, 0)),
        out_shape=jax.ShapeDtypeStruct((CHUNK, H), jnp.float32),
    )(edge, g1, w1c, w2, b1, b2, g_ln, b_ln)


def _scatter_body(ef0, ef1, ef2, ef3, ef4, dst_hbm, out_hbm,
                  idx_v, rows_v, zero_v, acc_sh, sem_l, sem_s):
    cid = lax.axis_index("c")
    sid = lax.axis_index("s")
    wid = sid * NC + cid

    def zrow(r, c):
        def zcol(q, c2):
            zero_v[r, pl.ds(q * 16, 16)] = jnp.zeros((16,), jnp.float32)
            return c2
        return lax.fori_loop(0, H // 16, zcol, c, unroll=True)

    lax.fori_loop(0, 16, zrow, 0)

    nb16 = N_GRID // 16

    def zput(t, c):
        bid = t * NS + sid

        @pl.when(bid < nb16)
        def _():
            pltpu.async_copy(zero_v, acc_sh.at[pl.ds(bid * 16, 16)],
                             sem_l.at[0])
        return c

    def zdrain(t, c):
        bid = t * NS + sid

        @pl.when(bid < nb16)
        def _():
            pltpu.make_async_copy(zero_v, acc_sh.at[pl.ds(0, 16)],
                                  sem_l.at[0]).wait()
        return c

    lax.fori_loop(0, (nb16 + NS - 1) // NS, zput, 0)
    lax.fori_loop(0, (nb16 + NS - 1) // NS, zdrain, 0)
    plsc.subcore_barrier()

    for c, ef_hbm in enumerate((ef0, ef1, ef2, ef3, ef4)):
        in0 = c * CHUNK + wid * EPWC
        ef0w = wid * EPWC

        def issue_load(i, b, ef_hbm=ef_hbm, in0=in0, ef0w=ef0w):
            pltpu.async_copy(dst_hbm.at[pl.ds(in0 + i * GBLK, GBLK)],
                             idx_v.at[b], sem_l.at[b])
            pltpu.async_copy(ef_hbm.at[pl.ds(ef0w + i * GBLK, GBLK)],
                             rows_v.at[b], sem_l.at[b])

        def wait_load(b, ef_hbm=ef_hbm):
            pltpu.make_async_copy(dst_hbm.at[pl.ds(0, GBLK)], idx_v.at[b],
                                  sem_l.at[b]).wait()
            pltpu.make_async_copy(ef_hbm.at[pl.ds(0, GBLK)], rows_v.at[b],
                                  sem_l.at[b]).wait()

        def wait_scat(b):
            pltpu.make_async_copy(rows_v.at[b], acc_sh.at[idx_v.at[b]],
                                  sem_s.at[b]).wait()

        for j in range(SNBUF - 1):
            issue_load(j, j)

        def block(i, carry, issue_load=issue_load, wait_load=wait_load,
                  wait_scat=wait_scat):
            b = lax.rem(i, SNBUF)
            wait_load(b)
            pltpu.async_copy(rows_v.at[b], acc_sh.at[idx_v.at[b]],
                             sem_s.at[b], add=True)

            @pl.when(i + SNBUF - 1 < NBLKC)
            def _():
                bn = lax.rem(i + SNBUF - 1, SNBUF)

                @pl.when(i >= 1)
                def _():
                    wait_scat(bn)
                issue_load(i + SNBUF - 1, bn)
            return carry

        lax.fori_loop(0, NBLKC, block, 0)
        for b in range(SNBUF):
            wait_scat(b)

    plsc.subcore_barrier()

    def put(t, c):
        bid = t * NS + sid

        @pl.when(bid < nb16)
        def _():
            b = lax.rem(t, SNBUF)

            @pl.when(t >= SNBUF)
            def _():
                pltpu.make_async_copy(rows_v.at[b, pl.ds(0, 16)],
                                      out_hbm.at[cid, pl.ds(0, 16)],
                                      sem_s.at[b]).wait()
            pltpu.sync_copy(acc_sh.at[pl.ds(bid * 16, 16)],
                            rows_v.at[b, pl.ds(0, 16)])
            pltpu.async_copy(rows_v.at[b, pl.ds(0, 16)],
                             out_hbm.at[cid, pl.ds(bid * 16, 16)],
                             sem_s.at[b])
        return c

    lax.fori_loop(0, (nb16 + NS - 1) // NS, put, 0)

    ntb_tile = lax.div(nb16 - sid + NS - 1, NS)

    def pdrain(t, c):
        bid = t * NS + sid

        @pl.when(jnp.logical_and(bid < nb16, t + SNBUF >= ntb_tile))
        def _():
            b = lax.rem(t, SNBUF)
            pltpu.make_async_copy(rows_v.at[b, pl.ds(0, 16)],
                                  out_hbm.at[cid, pl.ds(0, 16)],
                                  sem_s.at[b]).wait()
        return c

    lax.fori_loop(0, (nb16 + NS - 1) // NS, pdrain, 0)


@functools.cache
def _scatter_call():
    return pl.kernel(
        _scatter_body,
        out_type=jax.ShapeDtypeStruct((NC, N_GRID, H), jnp.float32),
        mesh=_sc_mesh(),
        compiler_params=pltpu.CompilerParams(needs_layout_passes=False),
        scratch_types=[
            pltpu.VMEM((SNBUF, GBLK), jnp.int32),
            pltpu.VMEM((SNBUF, GBLK, H), jnp.float32),
            pltpu.VMEM((16, H), jnp.float32),
            pltpu.VMEM_SHARED((N_GRID, H), jnp.float32),
            pltpu.SemaphoreType.DMA((SNBUF,)),
            pltpu.SemaphoreType.DMA((SNBUF,)),
        ],
    )


def _node_body(grid_ref, a0_ref, a1_ref, w1a_ref, w1b_ref, w2_ref, b1_ref,
               b2_ref, g_ln_ref, b_ln_ref, mask_ref, o_ref):
    x = grid_ref[...]
    ag = a0_ref[...] + a1_ref[...]
    h = jnp.dot(x, w1a_ref[...], preferred_element_type=jnp.float32)
    h = h + jnp.dot(ag, w1b_ref[...], preferred_element_type=jnp.float32)
    h = h + b1_ref[...]
    h = h * jax.nn.sigmoid(h)
    h = jnp.dot(h, w2_ref[...], preferred_element_type=jnp.float32)
    h = h + b2_ref[...]
    m = jnp.mean(h, axis=-1, keepdims=True)
    d = h - m
    v = jnp.mean(d * d, axis=-1, keepdims=True)
    ln = d * lax.rsqrt(v + 1e-5) * g_ln_ref[...] + b_ln_ref[...]
    o_ref[...] = x + ln * mask_ref[...]


def _node_mlp(grid, a0, a1, w1a, w1b, w2, b1, b2, g_ln, b_ln, mask):
    blk = 2000
    nblk = N_GRID // blk
    wspec = pl.BlockSpec((H, H), lambda i: (0, 0))
    vspec = pl.BlockSpec((1, H), lambda i: (0, 0))
    bspec = pl.BlockSpec((blk, H), lambda i: (i, 0))
    return pl.pallas_call(
        _node_body,
        grid=(nblk,),
        in_specs=[bspec, bspec, bspec, wspec, wspec, wspec, vspec, vspec,
                  vspec, vspec, pl.BlockSpec((blk, 1), lambda i: (i, 0))],
        out_specs=bspec,
        out_shape=jax.ShapeDtypeStruct((N_GRID, H), jnp.float32),
    )(grid, a0, a1, w1a, w1b, w2, b1, b2, g_ln, b_ln, mask)


def kernel(mesh2grid_edge_features, grid_node_features, mesh_node_features,
           halo_idx, dst_indices, src_indices, num_local,
           eW1, eb1, eW2, eb2, eg, ebb,
           nW1, nb1, nW2, nb2, ng, nbb):
    a_tab, b_tab = _compute_ab(mesh_node_features, grid_node_features,
                               eW1[:H], eW1[H:2 * H])

    w1c = eW1[2 * H:]
    eb1r, eb2r = eb1.reshape(1, H), eb2.reshape(1, H)
    egr, ebbr = eg.reshape(1, H), ebb.reshape(1, H)
    e_chunks = []
    for c in range(NCHUNK):
        g1 = _gather_call(c)(a_tab, b_tab, halo_idx, src_indices,
                             dst_indices)
        e_chunks.append(_edge_mlp(c, mesh2grid_edge_features, g1, w1c, eW2,
                                  eb1r, eb2r, egr, ebbr))

    agg = _scatter_call()(*e_chunks, dst_indices)

    mask = (jnp.arange(N_GRID, dtype=jnp.int32)[:, None]
            < num_local).astype(jnp.float32)
    return _node_mlp(grid_node_features, agg[0], agg[1],
                     nW1[:H], nW1[H:], nW2,
                     nb1.reshape(1, H), nb2.reshape(1, H),
                     ng.reshape(1, H), nbb.reshape(1, H), mask)


# speedup vs baseline: 1.0298x; 1.0283x over previous
import functools

import jax
import jax.numpy as jnp
from jax import lax
from jax.experimental import pallas as pl
from jax.experimental.pallas import tpu as pltpu
from jax.experimental.pallas import tpu_sc as plsc

H = 128
N_MESH = 10000
N_GRID = 10000
N_HALO = 2048
E = 320000

NC = 2
NS = 16
NW = NC * NS
GBLK = 80
NCHUNK = 5
CHUNK = E // NCHUNK
EPWC = CHUNK // NW
NBLKC = EPWC // GBLK

NBUF = 5
SNBUF = 3


@functools.cache
def _sc_mesh():
    return plsc.VectorSubcoreMesh(core_axis_name="c", subcore_axis_name="s",
                                  num_cores=NC, num_subcores=NS)


def _ab_body(mesh_ref, grid_ref, w1a_ref, w1b_ref, a_ref, b_ref):
    a_ref[...] = jnp.dot(mesh_ref[...], w1a_ref[...],
                         preferred_element_type=jnp.float32)
    b_ref[...] = jnp.dot(grid_ref[...], w1b_ref[...],
                         preferred_element_type=jnp.float32)


def _compute_ab(mesh, grid, w1a, w1b):
    blk = 2000
    nblk = N_MESH // blk
    return pl.pallas_call(
        _ab_body,
        grid=(nblk,),
        in_specs=[
            pl.BlockSpec((blk, H), lambda i: (i, 0)),
            pl.BlockSpec((blk, H), lambda i: (i, 0)),
            pl.BlockSpec((H, H), lambda i: (0, 0)),
            pl.BlockSpec((H, H), lambda i: (0, 0)),
        ],
        out_specs=[
            pl.BlockSpec((blk, H), lambda i: (i, 0)),
            pl.BlockSpec((blk, H), lambda i: (i, 0)),
        ],
        out_shape=[
            jax.ShapeDtypeStruct((N_MESH, H), jnp.float32),
            jax.ShapeDtypeStruct((N_GRID, H), jnp.float32),
        ],
    )(mesh, grid, w1a, w1b)


def _make_gather_body(coff):
  def _gather_body(a_hbm, b_hbm, halo_hbm, src_hbm, dst_hbm, out_hbm,
                     halo_v, src_v, dst_v, rows_v, sem_i, sem_a, sem_b, sem_w):
      wid = lax.axis_index("s") * NC + lax.axis_index("c")
      pltpu.sync_copy(halo_hbm, halo_v)
      in0 = coff + wid * EPWC
      base0 = wid * EPWC

      def issue_idx(i, b):
          base = in0 + i * GBLK
          pltpu.async_copy(src_hbm.at[pl.ds(base, GBLK)], src_v.at[b],
                           sem_i.at[b])
          pltpu.async_copy(dst_hbm.at[pl.ds(base, GBLK)], dst_v.at[b],
                           sem_i.at[b])

      def wait_idx(b):
          pltpu.make_async_copy(src_hbm.at[pl.ds(0, GBLK)], src_v.at[b],
                                sem_i.at[b]).wait()
          pltpu.make_async_copy(dst_hbm.at[pl.ds(0, GBLK)], dst_v.at[b],
                                sem_i.at[b]).wait()

      def remap(b):
          def step(j, c):
              s = src_v[b, pl.ds(j * 16, 16)]
              m = s >= N_MESH
              hidx = jnp.where(m, s - N_MESH, 0)
              hv = plsc.load_gather(halo_v, [hidx])
              src_v[b, pl.ds(j * 16, 16)] = jnp.where(m, hv, s)
              return c

          lax.fori_loop(0, GBLK // 16, step, 0, unroll=True)

      def wait_rows(b, sem):
          pltpu.make_async_copy(a_hbm.at[src_v.at[b]], rows_v.at[b],
                                sem.at[b]).wait()

      issue_idx(0, 0)
      issue_idx(1, 1)

      def step(i, carry):
          b = lax.rem(i, NBUF)
          wait_idx(b)
          remap(b)

          @pl.when(i >= NBUF)
          def _():
              pltpu.make_async_copy(rows_v.at[b],
                                    out_hbm.at[pl.ds(base0, GBLK)],
                                    sem_w.at[b]).wait()

          pltpu.async_copy(a_hbm.at[src_v.at[b]], rows_v.at[b], sem_a.at[b])

          @pl.when(i >= 1)
          def _():
              b1 = lax.rem(i - 1, NBUF)
              wait_rows(b1, sem_a)
              pltpu.async_copy(b_hbm.at[dst_v.at[b1]], rows_v.at[b1],
                               sem_b.at[b1], add=True)

          @pl.when(i >= 2)
          def _():
              b2 = lax.rem(i - 2, NBUF)
              wait_rows(b2, sem_b)
              base2 = base0 + (i - 2) * GBLK
              pltpu.async_copy(rows_v.at[b2], out_hbm.at[pl.ds(base2, GBLK)],
                               sem_w.at[b2])

          @pl.when(i + 2 < NBLKC)
          def _():
              issue_idx(i + 2, lax.rem(i + 2, NBUF))
          return carry

      lax.fori_loop(0, NBLKC, step, 0)

      bL = (NBLKC - 1) % NBUF
      b2 = (NBLKC - 2) % NBUF
      wait_rows(bL, sem_a)
      pltpu.async_copy(b_hbm.at[dst_v.at[bL]], rows_v.at[bL], sem_b.at[bL],
                       add=True)
      wait_rows(b2, sem_b)
      pltpu.async_copy(rows_v.at[b2],
                       out_hbm.at[pl.ds(base0 + (NBLKC - 2) * GBLK, GBLK)],
                       sem_w.at[b2])
      wait_rows(bL, sem_b)
      pltpu.async_copy(rows_v.at[bL],
                       out_hbm.at[pl.ds(base0 + (NBLKC - 1) * GBLK, GBLK)],
                       sem_w.at[bL])
      for b in range(NBUF):
          pltpu.make_async_copy(rows_v.at[b], out_hbm.at[pl.ds(base0, GBLK)],
                                sem_w.at[b]).wait()


  return _gather_body


@functools.cache
def _gather_call(c):
    return pl.kernel(
        _make_gather_body(c * CHUNK),
        out_type=jax.ShapeDtypeStruct((CHUNK, H), jnp.float32),
        mesh=_sc_mesh(),
        compiler_params=pltpu.CompilerParams(needs_layout_passes=False),
        scratch_types=[
            pltpu.VMEM((N_HALO,), jnp.int32),
            pltpu.VMEM((NBUF, GBLK), jnp.int32),
            pltpu.VMEM((NBUF, GBLK), jnp.int32),
            pltpu.VMEM((NBUF, GBLK, H), jnp.float32),
            pltpu.SemaphoreType.DMA((NBUF,)),
            pltpu.SemaphoreType.DMA((NBUF,)),
            pltpu.SemaphoreType.DMA((NBUF,)),
            pltpu.SemaphoreType.DMA((NBUF,)),
        ],
    )


def _edge_body(x_ref, g_ref, w1c_ref, w2_ref, b1_ref, b2_ref, g_ln_ref,
               b_ln_ref, o_ref):
    x = x_ref[...]
    h = jnp.dot(x, w1c_ref[...], preferred_element_type=jnp.float32)
    h = h + g_ref[...] + b1_ref[...]
    h = h * jax.nn.sigmoid(h)
    h = jnp.dot(h, w2_ref[...], preferred_element_type=jnp.float32)
    h = h + b2_ref[...]
    m = jnp.mean(h, axis=-1, keepdims=True)
    d = h - m
    v = jnp.mean(d * d, axis=-1, keepdims=True)
    ln = d * lax.rsqrt(v + 1e-5) * g_ln_ref[...] + b_ln_ref[...]
    o_ref[...] = x + ln


_EBLK = 6400
_EBPC = CHUNK // _EBLK


def _edge_mlp(c, edge, g1, w1c, w2, b1, b2, g_ln, b_ln):
    cb0 = c * _EBPC
    wspec = pl.BlockSpec((H, H), lambda i: (0, 0))
    vspec = pl.BlockSpec((1, H), lambda i: (0, 0))
    return pl.pallas_call(
        _edge_body,
        grid=(_EBPC,),
        in_specs=[
            pl.BlockSpec((_EBLK, H), lambda i: (cb0 + i, 0)),
            pl.BlockSpec((_EBLK, H), lambda i: (i, 0)),
            wspec, wspec, vspec, vspec, vspec, vspec,
        ],
        out_specs=pl.BlockSpec((_EBLK, H), lambda i: (i, 0)),
        out_shape=jax.ShapeDtypeStruct((CHUNK, H), jnp.float32),
    )(edge, g1, w1c, w2, b1, b2, g_ln, b_ln)


def _scatter_body(ef0, ef1, ef2, ef3, ef4, dst_hbm, out_hbm,
                  idx_v, rows_v, zero_v, acc_sh, sem_l, sem_s):
    cid = lax.axis_index("c")
    sid = lax.axis_index("s")
    wid = sid * NC + cid

    def zrow(r, c):
        def zcol(q, c2):
            zero_v[r, pl.ds(q * 16, 16)] = jnp.zeros((16,), jnp.float32)
            return c2
        return lax.fori_loop(0, H // 16, zcol, c, unroll=True)

    lax.fori_loop(0, 16, zrow, 0)

    nb16 = N_GRID // 16

    def zput(t, c):
        bid = t * NS + sid

        @pl.when(bid < nb16)
        def _():
            pltpu.async_copy(zero_v, acc_sh.at[pl.ds(bid * 16, 16)],
                             sem_l.at[0])
        return c

    def zdrain(t, c):
        bid = t * NS + sid

        @pl.when(bid < nb16)
        def _():
            pltpu.make_async_copy(zero_v, acc_sh.at[pl.ds(0, 16)],
                                  sem_l.at[0]).wait()
        return c

    lax.fori_loop(0, (nb16 + NS - 1) // NS, zput, 0)
    lax.fori_loop(0, (nb16 + NS - 1) // NS, zdrain, 0)
    plsc.subcore_barrier()

    for c, ef_hbm in enumerate((ef0, ef1, ef2, ef3, ef4)):
        in0 = c * CHUNK + wid * EPWC
        ef0w = wid * EPWC

        def issue_load(i, b, ef_hbm=ef_hbm, in0=in0, ef0w=ef0w):
            pltpu.async_copy(dst_hbm.at[pl.ds(in0 + i * GBLK, GBLK)],
                             idx_v.at[b], sem_l.at[b])
            pltpu.async_copy(ef_hbm.at[pl.ds(ef0w + i * GBLK, GBLK)],
                             rows_v.at[b], sem_l.at[b])

        def wait_load(b, ef_hbm=ef_hbm):
            pltpu.make_async_copy(dst_hbm.at[pl.ds(0, GBLK)], idx_v.at[b],
                                  sem_l.at[b]).wait()
            pltpu.make_async_copy(ef_hbm.at[pl.ds(0, GBLK)], rows_v.at[b],
                                  sem_l.at[b]).wait()

        def wait_scat(b):
            pltpu.make_async_copy(rows_v.at[b], acc_sh.at[idx_v.at[b]],
                                  sem_s.at[b]).wait()

        for j in range(SNBUF - 1):
            issue_load(j, j)

        def block(i, carry, issue_load=issue_load, wait_load=wait_load,
                  wait_scat=wait_scat):
            b = lax.rem(i, SNBUF)
            wait_load(b)
            pltpu.async_copy(rows_v.at[b], acc_sh.at[idx_v.at[b]],
                             sem_s.at[b], add=True)

            @pl.when(i + SNBUF - 1 < NBLKC)
            def _():
                bn = lax.rem(i + SNBUF - 1, SNBUF)

                @pl.when(i >= 1)
                def _():
                    wait_scat(bn)
                issue_load(i + SNBUF - 1, bn)
            return carry

        lax.fori_loop(0, NBLKC, block, 0)
        for b in range(SNBUF):
            wait_scat(b)

    plsc.subcore_barrier()

    def put(t, c):
        bid = t * NS + sid

        @pl.when(bid < nb16)
        def _():
            b = lax.rem(t, SNBUF)

            @pl.when(t >= SNBUF)
            def _():
                pltpu.make_async_copy(rows_v.at[b, pl.ds(0, 16)],
                                      out_hbm.at[cid, pl.ds(0, 16)],
                                      sem_s.at[b]).wait()
            pltpu.sync_copy(acc_sh.at[pl.ds(bid * 16, 16)],
                            rows_v.at[b, pl.ds(0, 16)])
            pltpu.async_copy(rows_v.at[b, pl.ds(0, 16)],
                             out_hbm.at[cid, pl.ds(bid * 16, 16)],
                             sem_s.at[b])
        return c

    lax.fori_loop(0, (nb16 + NS - 1) // NS, put, 0)

    ntb_tile = lax.div(nb16 - sid + NS - 1, NS)

    def pdrain(t, c):
        bid = t * NS + sid

        @pl.when(jnp.logical_and(bid < nb16, t + SNBUF >= ntb_tile))
        def _():
            b = lax.rem(t, SNBUF)
            pltpu.make_async_copy(rows_v.at[b, pl.ds(0, 16)],
                                  out_hbm.at[cid, pl.ds(0, 16)],
                                  sem_s.at[b]).wait()
        return c

    lax.fori_loop(0, (nb16 + NS - 1) // NS, pdrain, 0)


@functools.cache
def _scatter_call():
    return pl.kernel(
        _scatter_body,
        out_type=jax.ShapeDtypeStruct((NC, N_GRID, H), jnp.float32),
        mesh=_sc_mesh(),
        compiler_params=pltpu.CompilerParams(needs_layout_passes=False),
        scratch_types=[
            pltpu.VMEM((SNBUF, GBLK), jnp.int32),
            pltpu.VMEM((SNBUF, GBLK, H), jnp.float32),
            pltpu.VMEM((16, H), jnp.float32),
            pltpu.VMEM_SHARED((N_GRID, H), jnp.float32),
            pltpu.SemaphoreType.DMA((SNBUF,)),
            pltpu.SemaphoreType.DMA((SNBUF,)),
        ],
    )


def _node_body(grid_ref, a0_ref, a1_ref, w1a_ref, w1b_ref, w2_ref, b1_ref,
               b2_ref, g_ln_ref, b_ln_ref, mask_ref, o_ref):
    x = grid_ref[...]
    ag = a0_ref[...] + a1_ref[...]
    h = jnp.dot(x, w1a_ref[...], preferred_element_type=jnp.float32)
    h = h + jnp.dot(ag, w1b_ref[...], preferred_element_type=jnp.float32)
    h = h + b1_ref[...]
    h = h * jax.nn.sigmoid(h)
    h = jnp.dot(h, w2_ref[...], preferred_element_type=jnp.float32)
    h = h + b2_ref[...]
    m = jnp.mean(h, axis=-1, keepdims=True)
    d = h - m
    v = jnp.mean(d * d, axis=-1, keepdims=True)
    ln = d * lax.rsqrt(v + 1e-5) * g_ln_ref[...] + b_ln_ref[...]
    o_ref[...] = x + ln * mask_ref[...]


def _node_mlp(grid, a0, a1, w1a, w1b, w2, b1, b2, g_ln, b_ln, mask):
    blk = 2000
    nblk = N_GRID // blk
    wspec = pl.BlockSpec((H, H), lambda i: (0, 0))
    vspec = pl.BlockSpec((1, H), lambda i: (0, 0))
    bspec = pl.BlockSpec((blk, H), lambda i: (i, 0))
    return pl.pallas_call(
        _node_body,
        grid=(nblk,),
        in_specs=[bspec, bspec, bspec, wspec, wspec, wspec, vspec, vspec,
                  vspec, vspec, pl.BlockSpec((blk, 1), lambda i: (i, 0))],
        out_specs=bspec,
        out_shape=jax.ShapeDtypeStruct((N_GRID, H), jnp.float32),
    )(grid, a0, a1, w1a, w1b, w2, b1, b2, g_ln, b_ln, mask)


def kernel(mesh2grid_edge_features, grid_node_features, mesh_node_features,
           halo_idx, dst_indices, src_indices, num_local,
           eW1, eb1, eW2, eb2, eg, ebb,
           nW1, nb1, nW2, nb2, ng, nbb):
    a_tab, b_tab = _compute_ab(mesh_node_features, grid_node_features,
                               eW1[:H], eW1[H:2 * H])

    w1c = eW1[2 * H:]
    eb1r, eb2r = eb1.reshape(1, H), eb2.reshape(1, H)
    egr, ebbr = eg.reshape(1, H), ebb.reshape(1, H)
    e_chunks = []
    for c in range(NCHUNK):
        g1 = _gather_call(c)(a_tab, b_tab, halo_idx, src_indices,
                             dst_indices)
        e_chunks.append(_edge_mlp(c, mesh2grid_edge_features, g1, w1c, eW2,
                                  eb1r, eb2r, egr, ebbr))

    agg = _scatter_call()(*e_chunks, dst_indices)

    mask = (jnp.arange(N_GRID, dtype=jnp.int32)[:, None]
            < num_local).astype(jnp.float32)
    return _node_mlp(grid_node_features, agg[0], agg[1],
                     nW1[:H], nW1[H:], nW2,
                     nb1.reshape(1, H), nb2.reshape(1, H),
                     ng.reshape(1, H), nbb.reshape(1, H), mask)
